# baseline (speedup 1.0000x reference)
import functools

import jax
import jax.numpy as jnp
from jax import lax
from jax.experimental import pallas as pl
from jax.experimental.pallas import tpu as pltpu
from jax.experimental.pallas import tpu_sc as plsc

_BATCH = 16384
_BM = 2048
_NC, _NS = 2, 16
_NW = _NC * _NS
_BPW = _BATCH // _NW


def _sc_gather(table, idx):
    mesh = plsc.VectorSubcoreMesh(core_axis_name="c", subcore_axis_name="s")

    @functools.partial(
        pl.kernel,
        mesh=mesh,
        out_type=jax.ShapeDtypeStruct((_BATCH, 128), jnp.float32),
        scratch_types=[
            pltpu.VMEM((_BPW,), jnp.int32),
            pltpu.VMEM((_BPW, 128), jnp.float32),
            pltpu.SemaphoreType.DMA,
        ],
    )
    def k(table_hbm, idx_hbm, out_hbm, idx_v, rows_v, sem):
        wid = lax.axis_index("s") * _NC + lax.axis_index("c")
        base = wid * _BPW
        pltpu.sync_copy(idx_hbm.at[pl.ds(base, _BPW)], idx_v)
        pltpu.async_copy(table_hbm.at[idx_v], rows_v, sem).wait()
        pltpu.sync_copy(rows_v, out_hbm.at[pl.ds(base, _BPW)])

    return k(table, idx)


def _mlp_body(f_ref, g_ref, w1_ref, w2_ref, b2_ref, w3d_ref, b3d_ref,
              wd2_ref, bd2_ref, emb_ref, rec_ref):
    f16 = f_ref[...].astype(jnp.bfloat16)
    h1 = lax.dot_general(f16, w1_ref[...], (((1,), (0,)), ((), ())),
                         preferred_element_type=jnp.float32)
    h1 = jnp.maximum(h1 + g_ref[:, :32], 0.0).astype(jnp.bfloat16)
    h2 = lax.dot_general(h1, w2_ref[...], (((1,), (0,)), ((), ())),
                         preferred_element_type=jnp.float32)
    h2 = jnp.maximum(h2 + b2_ref[...], 0.0).astype(jnp.bfloat16)
    ed = lax.dot_general(h2, w3d_ref[...], (((1,), (0,)), ((), ())),
                         preferred_element_type=jnp.float32) + b3d_ref[...]
    emb_ref[...] = ed[:, :16]
    d = jnp.maximum(ed[:, 16:], 0.0).astype(jnp.bfloat16)
    rec_ref[...] = lax.dot_general(d, wd2_ref[...], (((1,), (0,)), ((), ())),
                                   preferred_element_type=jnp.float32) + bd2_ref[...]


def _mlp(features, G, W1a, W2, b2, W3d, b3d, Wd2, bd2):
    nb = _BATCH // _BM
    full = lambda shape: pl.BlockSpec(shape, lambda i: (0, 0))
    return pl.pallas_call(
        _mlp_body,
        grid=(nb,),
        in_specs=[
            pl.BlockSpec((_BM, 128), lambda i: (i, 0)),
            pl.BlockSpec((_BM, 128), lambda i: (i, 0)),
            full((128, 32)),
            full((32, 16)),
            full((1, 16)),
            full((16, 32)),
            full((1, 32)),
            full((16, 128)),
            full((1, 128)),
        ],
        out_specs=[
            pl.BlockSpec((_BM, 16), lambda i: (i, 0)),
            pl.BlockSpec((_BM, 128), lambda i: (i, 0)),
        ],
        out_shape=[
            jax.ShapeDtypeStruct((_BATCH, 16), jnp.float32),
            jax.ShapeDtypeStruct((_BATCH, 128), jnp.float32),
        ],
    )(features, G, W1a, W2, b2, W3d, b3d, Wd2, bd2)


def kernel(features, position_idx, pos_emb, W1, b1, W2, b2, W3, b3,
           Wd1, bd1, Wd2, bd2):
    T = pos_emb @ W1[128:] + b1
    T = jnp.zeros((8, 128), jnp.float32).at[:T.shape[0], :32].set(T)
    W1a = W1[:128].astype(jnp.bfloat16)
    W3d = jnp.concatenate([W3, W3 @ Wd1], axis=1).astype(jnp.bfloat16)
    b3d = jnp.concatenate([b3, b3 @ Wd1 + bd1])[None, :]

    G = _sc_gather(T, position_idx)

    emb, rec = _mlp(features, G, W1a, W2.astype(jnp.bfloat16), b2[None, :],
                    W3d, b3d, Wd2.astype(jnp.bfloat16), bd2[None, :])
    return (emb, rec)


# speedup vs baseline: 4.0162x; 4.0162x over previous
import jax
import jax.numpy as jnp
from jax import lax
from jax.experimental import pallas as pl

_BATCH = 16384
_BM = 2048


def _mlp_body(f_ref, idx_ref, t_ref, w1_ref, w2_ref, b2_ref, w3d_ref, b3d_ref,
              wd2_ref, bd2_ref, emb_ref, rec_ref):
    onehot = (idx_ref[...] == lax.broadcasted_iota(jnp.int32, (_BM, 8), 1)
              ).astype(jnp.bfloat16)
    g = lax.dot_general(onehot, t_ref[...], (((1,), (0,)), ((), ())),
                        preferred_element_type=jnp.float32)
    f16 = f_ref[...].astype(jnp.bfloat16)
    h1 = lax.dot_general(f16, w1_ref[...], (((1,), (0,)), ((), ())),
                         preferred_element_type=jnp.float32)
    h1 = jnp.maximum(h1 + g, 0.0).astype(jnp.bfloat16)
    h2 = lax.dot_general(h1, w2_ref[...], (((1,), (0,)), ((), ())),
                         preferred_element_type=jnp.float32)
    h2 = jnp.maximum(h2 + b2_ref[...], 0.0).astype(jnp.bfloat16)
    ed = lax.dot_general(h2, w3d_ref[...], (((1,), (0,)), ((), ())),
                         preferred_element_type=jnp.float32) + b3d_ref[...]
    emb_ref[...] = ed[:, :16]
    d = jnp.maximum(ed[:, 16:], 0.0).astype(jnp.bfloat16)
    rec_ref[...] = lax.dot_general(d, wd2_ref[...], (((1,), (0,)), ((), ())),
                                   preferred_element_type=jnp.float32) + bd2_ref[...]


def kernel(features, position_idx, pos_emb, W1, b1, W2, b2, W3, b3,
           Wd1, bd1, Wd2, bd2):
    T = pos_emb @ W1[128:] + b1
    T8 = jnp.zeros((8, 32), jnp.float32).at[:T.shape[0]].set(T).astype(jnp.bfloat16)
    W1a = W1[:128].astype(jnp.bfloat16)
    W3d = jnp.concatenate([W3, W3 @ Wd1], axis=1).astype(jnp.bfloat16)
    b3d = jnp.concatenate([b3, b3 @ Wd1 + bd1])[None, :]

    nb = _BATCH // _BM
    full = lambda shape: pl.BlockSpec(shape, lambda i: (0, 0))
    emb, rec = pl.pallas_call(
        _mlp_body,
        grid=(nb,),
        in_specs=[
            pl.BlockSpec((_BM, 128), lambda i: (i, 0)),
            pl.BlockSpec((_BM, 1), lambda i: (i, 0)),
            full((8, 32)),
            full((128, 32)),
            full((32, 16)),
            full((1, 16)),
            full((16, 32)),
            full((1, 32)),
            full((16, 128)),
            full((1, 128)),
        ],
        out_specs=[
            pl.BlockSpec((_BM, 16), lambda i: (i, 0)),
            pl.BlockSpec((_BM, 128), lambda i: (i, 0)),
        ],
        out_shape=[
            jax.ShapeDtypeStruct((_BATCH, 16), jnp.float32),
            jax.ShapeDtypeStruct((_BATCH, 128), jnp.float32),
        ],
    )(features, position_idx[:, None], T8, W1a, W2.astype(jnp.bfloat16),
      b2[None, :], W3d, b3d, Wd2.astype(jnp.bfloat16), bd2[None, :])
    return (emb, rec)


# speedup vs baseline: 4.2636x; 1.0616x over previous
import jax
import jax.numpy as jnp
from jax import lax
from jax.experimental import pallas as pl

_BATCH = 16384
_BM = 4096


def _mlp_body(f_ref, idx_ref, t_ref, w1_ref, w2_ref, b2_ref, w3d_ref, b3d_ref,
              wd2_ref, bd2_ref, emb_ref, rec_ref):
    onehot = (idx_ref[...] == lax.broadcasted_iota(jnp.int32, (_BM, 8), 1)
              ).astype(jnp.bfloat16)
    g = lax.dot_general(onehot, t_ref[...], (((1,), (0,)), ((), ())),
                        preferred_element_type=jnp.float32)
    f16 = f_ref[...].astype(jnp.bfloat16)
    h1 = lax.dot_general(f16, w1_ref[...], (((1,), (0,)), ((), ())),
                         preferred_element_type=jnp.float32)
    h1 = jnp.maximum(h1 + g, 0.0).astype(jnp.bfloat16)
    h2 = lax.dot_general(h1, w2_ref[...], (((1,), (0,)), ((), ())),
                         preferred_element_type=jnp.float32)
    h2 = jnp.maximum(h2 + b2_ref[...], 0.0).astype(jnp.bfloat16)
    ed = lax.dot_general(h2, w3d_ref[...], (((1,), (0,)), ((), ())),
                         preferred_element_type=jnp.float32) + b3d_ref[...]
    emb_ref[...] = ed[:, :16]
    d = jnp.maximum(ed[:, 16:], 0.0).astype(jnp.bfloat16)
    rec_ref[...] = lax.dot_general(d, wd2_ref[...], (((1,), (0,)), ((), ())),
                                   preferred_element_type=jnp.float32) + bd2_ref[...]


def kernel(features, position_idx, pos_emb, W1, b1, W2, b2, W3, b3,
           Wd1, bd1, Wd2, bd2):
    T = pos_emb @ W1[128:] + b1
    T8 = jnp.zeros((8, 32), jnp.float32).at[:T.shape[0]].set(T).astype(jnp.bfloat16)
    W1a = W1[:128].astype(jnp.bfloat16)
    W3d = jnp.concatenate([W3, W3 @ Wd1], axis=1).astype(jnp.bfloat16)
    b3d = jnp.concatenate([b3, b3 @ Wd1 + bd1])[None, :]

    nb = _BATCH // _BM
    full = lambda shape: pl.BlockSpec(shape, lambda i: (0, 0))
    emb, rec = pl.pallas_call(
        _mlp_body,
        grid=(nb,),
        in_specs=[
            pl.BlockSpec((_BM, 128), lambda i: (i, 0)),
            pl.BlockSpec((_BM, 1), lambda i: (i, 0)),
            full((8, 32)),
            full((128, 32)),
            full((32, 16)),
            full((1, 16)),
            full((16, 32)),
            full((1, 32)),
            full((16, 128)),
            full((1, 128)),
        ],
        out_specs=[
            pl.BlockSpec((_BM, 16), lambda i: (i, 0)),
            pl.BlockSpec((_BM, 128), lambda i: (i, 0)),
        ],
        out_shape=[
            jax.ShapeDtypeStruct((_BATCH, 16), jnp.float32),
            jax.ShapeDtypeStruct((_BATCH, 128), jnp.float32),
        ],
    )(features, position_idx[:, None], T8, W1a, W2.astype(jnp.bfloat16),
      b2[None, :], W3d, b3d, Wd2.astype(jnp.bfloat16), bd2[None, :])
    return (emb, rec)


# speedup vs baseline: 4.5623x; 1.0701x over previous
import jax
import jax.numpy as jnp
from jax import lax
from jax.experimental import pallas as pl

_BATCH = 16384
_BM = 8192


def _mlp_body(f_ref, idx_ref, t_ref, w1_ref, w2_ref, b2_ref, w3d_ref, b3d_ref,
              wd2_ref, bd2_ref, emb_ref, rec_ref):
    iota = lax.broadcasted_iota(jnp.int32, (_BM, 8), 1).astype(jnp.bfloat16)
    onehot = (idx_ref[...] == iota).astype(jnp.bfloat16)
    g = lax.dot_general(onehot, t_ref[...], (((1,), (0,)), ((), ())),
                        preferred_element_type=jnp.float32)
    f16 = f_ref[...].astype(jnp.bfloat16)
    h1 = lax.dot_general(f16, w1_ref[...], (((1,), (0,)), ((), ())),
                         preferred_element_type=jnp.float32)
    h1 = jnp.maximum(h1 + g, 0.0).astype(jnp.bfloat16)
    h2 = lax.dot_general(h1, w2_ref[...], (((1,), (0,)), ((), ())),
                         preferred_element_type=jnp.float32)
    h2 = jnp.maximum(h2 + b2_ref[...], 0.0).astype(jnp.bfloat16)
    ed = lax.dot_general(h2, w3d_ref[...], (((1,), (0,)), ((), ())),
                         preferred_element_type=jnp.float32) + b3d_ref[...]
    emb_ref[...] = ed[:, :16]
    d = jnp.maximum(ed[:, 16:], 0.0).astype(jnp.bfloat16)
    rec_ref[...] = lax.dot_general(d, wd2_ref[...], (((1,), (0,)), ((), ())),
                                   preferred_element_type=jnp.float32) + bd2_ref[...]


def kernel(features, position_idx, pos_emb, W1, b1, W2, b2, W3, b3,
           Wd1, bd1, Wd2, bd2):
    T = pos_emb @ W1[128:] + b1
    T8 = jnp.zeros((8, 32), jnp.float32).at[:T.shape[0]].set(T).astype(jnp.bfloat16)
    W1a = W1[:128].astype(jnp.bfloat16)
    W3d = jnp.concatenate([W3, W3 @ Wd1], axis=1).astype(jnp.bfloat16)
    b3d = jnp.concatenate([b3, b3 @ Wd1 + bd1])[None, :]
    idxb = jnp.broadcast_to(position_idx.astype(jnp.bfloat16)[:, None],
                            (_BATCH, 8))

    nb = _BATCH // _BM
    full = lambda shape: pl.BlockSpec(shape, lambda i: (0, 0))
    emb, rec = pl.pallas_call(
        _mlp_body,
        grid=(nb,),
        in_specs=[
            pl.BlockSpec((_BM, 128), lambda i: (i, 0)),
            pl.BlockSpec((_BM, 8), lambda i: (i, 0)),
            full((8, 32)),
            full((128, 32)),
            full((32, 16)),
            full((1, 16)),
            full((16, 32)),
            full((1, 32)),
            full((16, 128)),
            full((1, 128)),
        ],
        out_specs=[
            pl.BlockSpec((_BM, 16), lambda i: (i, 0)),
            pl.BlockSpec((_BM, 128), lambda i: (i, 0)),
        ],
        out_shape=[
            jax.ShapeDtypeStruct((_BATCH, 16), jnp.float32),
            jax.ShapeDtypeStruct((_BATCH, 128), jnp.float32),
        ],
    )(features, idxb, T8, W1a, W2.astype(jnp.bfloat16),
      b2[None, :], W3d, b3d, Wd2.astype(jnp.bfloat16), bd2[None, :])
    return (emb, rec)


# speedup vs baseline: 5.4377x; 1.1919x over previous
import jax
import jax.numpy as jnp
from jax import lax
from jax.experimental import pallas as pl

_BATCH = 16384
_BM = 8192


def _mlp_body(f_ref, idx_ref, t_ref, w1_ref, w2_ref, b2_ref, w3d_ref, b3d_ref,
              wd2_ref, bd2_ref, emb_ref, rec_ref):
    idxb = jnp.broadcast_to(idx_ref[...], (8, _BM))
    iota = lax.broadcasted_iota(jnp.int32, (8, _BM), 0)
    onehot_t = (idxb == iota).astype(jnp.bfloat16)
    g = lax.dot_general(onehot_t, t_ref[...], (((0,), (0,)), ((), ())),
                        preferred_element_type=jnp.float32)
    f16 = f_ref[...].astype(jnp.bfloat16)
    h1 = lax.dot_general(f16, w1_ref[...], (((1,), (0,)), ((), ())),
                         preferred_element_type=jnp.float32)
    h1 = jnp.maximum(h1 + g, 0.0).astype(jnp.bfloat16)
    h2 = lax.dot_general(h1, w2_ref[...], (((1,), (0,)), ((), ())),
                         preferred_element_type=jnp.float32)
    h2 = jnp.maximum(h2 + b2_ref[...], 0.0).astype(jnp.bfloat16)
    ed = lax.dot_general(h2, w3d_ref[...], (((1,), (0,)), ((), ())),
                         preferred_element_type=jnp.float32) + b3d_ref[...]
    emb_ref[...] = ed[:, :16]
    d = jnp.maximum(ed[:, 16:], 0.0).astype(jnp.bfloat16)
    rec_ref[...] = lax.dot_general(d, wd2_ref[...], (((1,), (0,)), ((), ())),
                                   preferred_element_type=jnp.float32) + bd2_ref[...]


def kernel(features, position_idx, pos_emb, W1, b1, W2, b2, W3, b3,
           Wd1, bd1, Wd2, bd2):
    T = pos_emb @ W1[128:] + b1
    T8 = jnp.zeros((8, 32), jnp.float32).at[:T.shape[0]].set(T).astype(jnp.bfloat16)
    W1a = W1[:128].astype(jnp.bfloat16)
    W3d = jnp.concatenate([W3, W3 @ Wd1], axis=1).astype(jnp.bfloat16)
    b3d = jnp.concatenate([b3, b3 @ Wd1 + bd1])[None, :]
    idx2d = position_idx.reshape(1, _BATCH)

    nb = _BATCH // _BM
    full = lambda shape: pl.BlockSpec(shape, lambda i: (0, 0))
    emb, rec = pl.pallas_call(
        _mlp_body,
        grid=(nb,),
        in_specs=[
            pl.BlockSpec((_BM, 128), lambda i: (i, 0)),
            pl.BlockSpec((1, _BM), lambda i: (0, i)),
            full((8, 32)),
            full((128, 32)),
            full((32, 16)),
            full((1, 16)),
            full((16, 32)),
            full((1, 32)),
            full((16, 128)),
            full((1, 128)),
        ],
        out_specs=[
            pl.BlockSpec((_BM, 16), lambda i: (i, 0)),
            pl.BlockSpec((_BM, 128), lambda i: (i, 0)),
        ],
        out_shape=[
            jax.ShapeDtypeStruct((_BATCH, 16), jnp.float32),
            jax.ShapeDtypeStruct((_BATCH, 128), jnp.float32),
        ],
    )(features, idx2d, T8, W1a, W2.astype(jnp.bfloat16),
      b2[None, :], W3d, b3d, Wd2.astype(jnp.bfloat16), bd2[None, :])
    return (emb, rec)
